# baseline (speedup 1.0000x reference)
import jax
import jax.numpy as jnp
from jax.experimental import pallas as pl
from jax.experimental.pallas import tpu as pltpu


_MB = 128


def _gconv_body(x_ref, adj_ref, wt_ref, wlt_ref, b_ref, out_ref, v_ref, sl_ref):
    k = wt_ref.shape[0]
    n = v_ref.shape[0]
    i = pl.program_id(0)

    @pl.when(i == 0)
    def _build_panels():
        w = wt_ref[:].T
        wl = wlt_ref[:].T
        x0 = x_ref[pl.ds(0, n), :]
        x1 = x_ref[pl.ds(n, n), :]
        b = b_ref[:].reshape(1, k)
        v_ref[:, :k] = jnp.dot(
            x0, w, preferred_element_type=jnp.float32
        ).astype(jnp.bfloat16)
        v_ref[:, k:] = jnp.dot(
            x1, w, preferred_element_type=jnp.float32
        ).astype(jnp.bfloat16)
        sl_ref[pl.ds(0, n), :k] = (
            jnp.dot(x0, wl, preferred_element_type=jnp.float32) + b
        )
        sl_ref[pl.ds(0, n), k:] = (
            jnp.dot(x1, wl, preferred_element_type=jnp.float32) + b
        )

    acc = jnp.dot(
        adj_ref[:].astype(jnp.bfloat16),
        v_ref[:],
        preferred_element_type=jnp.float32,
    )
    out_ref[:] = jnp.maximum(acc + sl_ref[pl.ds(i * _MB, _MB), :], 0.0).T


def kernel(inputs, adj_mat, weight, loop_weight, bias):
    batch, n, f = inputs.shape
    k = weight.shape[1]

    packed = pl.pallas_call(
        _gconv_body,
        grid=(pl.cdiv(n, _MB),),
        in_specs=[
            pl.BlockSpec((batch * n, f), lambda i: (0, 0)),
            pl.BlockSpec((_MB, n), lambda i: (i, 0)),
            pl.BlockSpec((k, f), lambda i: (0, 0)),
            pl.BlockSpec((k, f), lambda i: (0, 0)),
            pl.BlockSpec((k,), lambda i: (0,)),
        ],
        out_specs=pl.BlockSpec((batch * k, _MB), lambda i: (0, i)),
        out_shape=jax.ShapeDtypeStruct((batch * k, n), jnp.float32),
        scratch_shapes=[
            pltpu.VMEM((n, batch * k), jnp.bfloat16),
            pltpu.VMEM((pl.cdiv(n, _MB) * _MB, batch * k), jnp.float32),
        ],
    )(
        inputs.reshape(batch * n, f),
        adj_mat,
        weight.T,
        loop_weight.T,
        bias,
    )
    return jnp.transpose(packed.reshape(batch, k, n), (0, 2, 1))
